# Initial kernel scaffold; baseline (speedup 1.0000x reference)
import jax
import jax.numpy as jnp
from jax.experimental import pallas as pl


def kernel(hidden_states, topk_weights, topk_ids, w1, w3, w2):
    raise NotImplementedError("write your pallas kernel here")


# speedup vs baseline: 3.5705x; 3.5705x over previous
import jax
import jax.numpy as jnp
from jax.experimental import pallas as pl
from jax.experimental.pallas import tpu as pltpu

_FT = 512


def _moe_body(x_ref, tw_ref, ti_ref, w1_ref, w3_ref, w2_ref, out_ref):
    e = pl.program_id(0)
    f = pl.program_id(1)
    x = x_ref[...]
    g = jax.lax.dot_general(x, w1_ref[0], (((1,), (1,)), ((), ())),
                            preferred_element_type=jnp.float32)
    u = jax.lax.dot_general(x, w3_ref[0], (((1,), (1,)), ((), ())),
                            preferred_element_type=jnp.float32)
    h = (g * jax.nn.sigmoid(g)) * u
    y = jax.lax.dot_general(h, w2_ref[0], (((1,), (1,)), ((), ())),
                            preferred_element_type=jnp.float32)
    we = jnp.sum(tw_ref[...] * (ti_ref[...] == e).astype(jnp.float32), axis=1)
    contrib = y * we[:, None]

    @pl.when(jnp.logical_and(e == 0, f == 0))
    def _():
        out_ref[...] = contrib

    @pl.when(jnp.logical_or(e != 0, f != 0))
    def _():
        out_ref[...] = out_ref[...] + contrib


@jax.jit
def kernel(hidden_states, topk_weights, topk_ids, w1, w3, w2):
    T, D = hidden_states.shape
    E, DFF, _ = w1.shape
    K = topk_ids.shape[1]
    n_f = DFF // _FT
    return pl.pallas_call(
        _moe_body,
        grid=(E, n_f),
        in_specs=[
            pl.BlockSpec((T, D), lambda e, f: (0, 0)),
            pl.BlockSpec((T, K), lambda e, f: (0, 0)),
            pl.BlockSpec((T, K), lambda e, f: (0, 0)),
            pl.BlockSpec((1, _FT, D), lambda e, f: (e, f, 0)),
            pl.BlockSpec((1, _FT, D), lambda e, f: (e, f, 0)),
            pl.BlockSpec((1, D, _FT), lambda e, f: (e, 0, f)),
        ],
        out_specs=pl.BlockSpec((T, D), lambda e, f: (0, 0)),
        out_shape=jax.ShapeDtypeStruct((T, D), jnp.float32),
        compiler_params=pltpu.CompilerParams(
            dimension_semantics=("arbitrary", "arbitrary"),
        ),
    )(hidden_states, topk_weights, topk_ids, w1, w3, w2)
